# baseline (speedup 1.0000x reference)
import functools

import jax
import jax.numpy as jnp
from jax import lax
from jax.experimental import pallas as pl
from jax.experimental.pallas import tpu as pltpu
from jax.experimental.pallas import tpu_sc as plsc

NUM_EMB = 1000000
DIM = 32
BATCH = 4096
HIST = 50

NC = 2
NS = 16
NW = NC * NS
BAGS_PER_TILE = BATCH // NW
BAGS_PER_CHUNK = 2
CHUNK_ROWS = BAGS_PER_CHUNK * HIST
CHUNK_PAD = 104
CHUNKS = BAGS_PER_TILE // BAGS_PER_CHUNK
IDX_ROWS = BATCH * HIST // CHUNK_ROWS
ROWS_PER_TILE = IDX_ROWS // NW


def _bag_mean(buf, base):
    h0 = [buf[base + k, 0:16] for k in range(4)]
    h1 = [buf[base + k, 16:32] for k in range(4)]
    for l in range(4, HIST):
        h0[l & 3] = h0[l & 3] + buf[base + l, 0:16]
        h1[l & 3] = h1[l & 3] + buf[base + l, 16:32]
    s0 = (h0[0] + h0[1]) + (h0[2] + h0[3])
    s1 = (h1[0] + h1[1]) + (h1[2] + h1[3])
    inv = jnp.float32(1.0 / HIST)
    return s0 * inv, s1 * inv


def _sc_body(idx_hbm, table_hbm, out_hbm, idx_v, buf0, buf1, out_v,
             sem0, sem1):
    wid = lax.axis_index("s") * NC + lax.axis_index("c")
    row0 = wid * ROWS_PER_TILE

    pltpu.sync_copy(idx_hbm.at[pl.ds(row0, ROWS_PER_TILE), :], idx_v)

    bufs = (buf0, buf1)
    sems = (sem0, sem1)

    def _start(c, slot):
        pltpu.make_async_copy(
            table_hbm.at[idx_v.at[c]], bufs[slot], sems[slot]).start()

    def _wait(c, slot):
        pltpu.make_async_copy(
            table_hbm.at[idx_v.at[c]], bufs[slot], sems[slot]).wait()

    _start(0, 0)

    @pl.loop(0, CHUNKS, step=2)
    def _chunk(c):
        for b in range(2):
            cc = c + b

            @pl.when(cc + 1 < CHUNKS)
            def _():
                _start(cc + 1, 1 - b)

            _wait(cc, b)
            buf = bufs[b]
            for i in range(BAGS_PER_CHUNK):
                s0, s1 = _bag_mean(buf, i * HIST)
                r = cc * BAGS_PER_CHUNK + i
                out_v[r, 0:16] = s0
                out_v[r, 16:32] = s1

    pltpu.sync_copy(out_v, out_hbm.at[pl.ds(wid * BAGS_PER_TILE,
                                            BAGS_PER_TILE), :])


@jax.jit
def _sc_call(idx_p, weight):
    mesh = plsc.VectorSubcoreMesh(core_axis_name="c", subcore_axis_name="s")
    return pl.kernel(
        _sc_body,
        out_type=jax.ShapeDtypeStruct((BATCH, DIM), jnp.float32),
        mesh=mesh,
        compiler_params=pltpu.CompilerParams(use_tc_tiling_on_sc=False),
        scratch_types=[
            pltpu.VMEM((ROWS_PER_TILE, CHUNK_PAD), jnp.int32),
            pltpu.VMEM((CHUNK_PAD, DIM), jnp.float32),
            pltpu.VMEM((CHUNK_PAD, DIM), jnp.float32),
            pltpu.VMEM((BAGS_PER_TILE, DIM), jnp.float32),
            pltpu.SemaphoreType.DMA,
            pltpu.SemaphoreType.DMA,
        ],
    )(idx_p, weight)


def kernel(input, weight):
    idx = input.astype(jnp.int32).reshape(IDX_ROWS, CHUNK_ROWS)
    idx_p = jnp.pad(idx, ((0, 0), (0, CHUNK_PAD - CHUNK_ROWS)))
    return _sc_call(idx_p, weight)
